# baseline (speedup 1.0000x reference)
import functools
import jax
import jax.numpy as jnp
from jax import lax
from jax.experimental import pallas as pl
from jax.experimental.pallas import tpu as pltpu
from jax.experimental.pallas import tpu_sc as plsc

B, S, D = 4, 4096, 2048
NC, NS = 2, 16
NW = NC * NS
S_PER_W = S // NW
CS = 8
CD = 1024
NQ = D // CD
N_RCHUNK = S_PER_W // CS
N_UNIT = N_RCHUNK * NQ
NSLOT = 3


def _sc_add_body(x_hbm, emb_hbm, out_hbm, emb_v, x_v, *sems):
    in_sems = sems[:NSLOT]
    out_sems = sems[NSLOT:]
    wid = lax.axis_index("s") * NC + lax.axis_index("c")
    s_base = wid * S_PER_W

    def issue_in(row, col, slot):
        pltpu.async_copy(
            emb_hbm.at[pl.ds(row, CS), pl.ds(col, CD)], emb_v.at[slot],
            in_sems[slot],
        )
        for b in range(B):
            pltpu.async_copy(
                x_hbm.at[b, pl.ds(row, CS), pl.ds(col, CD)], x_v.at[slot, b],
                in_sems[slot],
            )

    def wait_in(row, col, slot):
        pltpu.make_async_copy(
            emb_hbm.at[pl.ds(row, CS), pl.ds(col, CD)], emb_v.at[slot],
            in_sems[slot],
        ).wait()
        for b in range(B):
            pltpu.make_async_copy(
                x_hbm.at[b, pl.ds(row, CS), pl.ds(col, CD)], x_v.at[slot, b],
                in_sems[slot],
            ).wait()

    def issue_out(row, col, slot):
        for b in range(B):
            pltpu.async_copy(
                x_v.at[slot, b], out_hbm.at[b, pl.ds(row, CS), pl.ds(col, CD)],
                out_sems[slot],
            )

    def wait_out(row, col, slot):
        for b in range(B):
            pltpu.make_async_copy(
                x_v.at[slot, b], out_hbm.at[b, pl.ds(row, CS), pl.ds(col, CD)],
                out_sems[slot],
            ).wait()

    def compute(slot):
        @plsc.parallel_loop(0, CD, step=16, unroll=2)
        def _(i):
            sl = pl.ds(i, 16)
            for r in range(CS):
                e = emb_v[slot, r, sl]
                for b in range(B):
                    plsc.addupdate(x_v.at[slot, b, r, sl], e)

    def rowcol(u_chunk, u_col):
        return s_base + u_chunk * CS, u_col * CD

    issue_in(s_base, 0, 0)

    def tt_body(tt, _):
        for k in range(2 * NSLOT):
            u_chunk = 3 * tt + k // 2
            row, col = rowcol(u_chunk, k % 2)
            slot = k % NSLOT
            nslot = (k + 1) % NSLOT
            if k >= 2:
                prow, pcol = rowcol(3 * tt + (k - 2) // 2, k % 2)
                wait_out(prow, pcol, nslot)
            else:
                @pl.when(tt >= 1)
                def _():
                    prow, pcol = rowcol(3 * tt - 1, k % 2)
                    wait_out(prow, pcol, nslot)
            nrow, ncol = rowcol(3 * tt + (k + 1) // 2, (k + 1) % 2)
            issue_in(nrow, ncol, nslot)
            wait_in(row, col, slot)
            compute(slot)
            issue_out(row, col, slot)
        return ()

    lax.fori_loop(0, N_UNIT // (2 * NSLOT), tt_body, ())

    r15 = s_base + (N_RCHUNK - 1) * CS
    r14 = s_base + (N_RCHUNK - 2) * CS
    wait_out(r14, 0, 1)
    issue_in(r15, CD, 1)
    wait_in(r15, 0, 0)
    compute(0)
    issue_out(r15, 0, 0)
    wait_out(r14, CD, 2)
    wait_in(r15, CD, 1)
    compute(1)
    issue_out(r15, CD, 1)
    wait_out(r15, 0, 0)
    wait_out(r15, CD, 1)


@functools.partial(
    pl.kernel,
    out_type=jax.ShapeDtypeStruct((B, S, D), jnp.float32),
    mesh=plsc.VectorSubcoreMesh(core_axis_name="c", subcore_axis_name="s"),
    scratch_types=[
        pltpu.VMEM((NSLOT, CS, CD), jnp.float32),
        pltpu.VMEM((NSLOT, B, CS, CD), jnp.float32),
    ]
    + [pltpu.SemaphoreType.DMA] * (2 * NSLOT),
)
def _sc_add(x_hbm, emb_hbm, out_hbm, emb_v, x_v, *sems):
    _sc_add_body(x_hbm, emb_hbm, out_hbm, emb_v, x_v, *sems)


def kernel(x, emb_table):
    return _sc_add(x, emb_table)
